# baseline (speedup 1.0000x reference)
import functools

import jax
import jax.numpy as jnp
from jax import lax
from jax.experimental import pallas as pl
from jax.experimental.pallas import tpu as pltpu
from jax.experimental.pallas import tpu_sc as plsc

N_SAMPLES = 100000
NPC_DIM = 128
BATCH = 4096
TEMP = 0.07
CONST = 1e-12

_NC = 2
_NS = 16
_NW = _NC * _NS
_BPW = BATCH // _NW

_TN = 5000
_NT = N_SAMPLES // _TN


@functools.cache
def _get_sc_gather():
    mesh = plsc.VectorSubcoreMesh(core_axis_name="c", subcore_axis_name="s")

    @functools.partial(
        pl.kernel,
        mesh=mesh,
        out_type=[
            jax.ShapeDtypeStruct((BATCH,), jnp.int32),
            jax.ShapeDtypeStruct((BATCH, NPC_DIM), jnp.float32),
            jax.ShapeDtypeStruct((BATCH, NPC_DIM), jnp.float32),
            jax.ShapeDtypeStruct((BATCH, NPC_DIM), jnp.float32),
        ],
        scratch_types=[
            pltpu.VMEM((_BPW,), jnp.int32),
            pltpu.VMEM((_BPW,), jnp.int32),
            pltpu.VMEM((_BPW,), jnp.int32),
            pltpu.VMEM((_BPW,), jnp.int32),
            pltpu.VMEM((_BPW, NPC_DIM), jnp.float32),
            pltpu.VMEM((_BPW, NPC_DIM), jnp.float32),
            pltpu.VMEM((_BPW, NPC_DIM), jnp.float32),
            pltpu.SemaphoreType.DMA,
            pltpu.SemaphoreType.DMA,
        ],
    )
    def _sc_gather(index_hbm, lni_hbm, memory_hbm, flag_hbm, neighbors_hbm,
                   flags_out, rows_self_out, rows_ln_out, rows_nb_out,
                   idx_v, lni_v, nbi_v, flg_v, rs_v, rl_v, rn_v, sem_a, sem_b):
        wid = lax.axis_index("s") * _NC + lax.axis_index("c")
        base = wid * _BPW
        pltpu.sync_copy(index_hbm.at[pl.ds(base, _BPW)], idx_v)
        pltpu.sync_copy(lni_hbm.at[pl.ds(base, _BPW)], lni_v)
        c_nb = pltpu.async_copy(neighbors_hbm.at[idx_v], nbi_v, sem_a)
        c_fl = pltpu.async_copy(flag_hbm.at[idx_v], flg_v, sem_b)
        c_rs = pltpu.async_copy(memory_hbm.at[idx_v], rs_v, sem_b)
        c_rl = pltpu.async_copy(memory_hbm.at[lni_v], rl_v, sem_b)
        c_nb.wait()
        c_rn = pltpu.async_copy(memory_hbm.at[nbi_v], rn_v, sem_b)
        c_fl.wait()
        c_rs.wait()
        c_rl.wait()
        c_rn.wait()
        pltpu.sync_copy(flg_v, flags_out.at[pl.ds(base, _BPW)])
        pltpu.sync_copy(rs_v, rows_self_out.at[pl.ds(base, _BPW)])
        pltpu.sync_copy(rl_v, rows_ln_out.at[pl.ds(base, _BPW)])
        pltpu.sync_copy(rn_v, rows_nb_out.at[pl.ds(base, _BPW)])

    return _sc_gather


_LOG2E = 1.4426950408889634


def _stream_body(zp_ref, mem_ref, znt_ref, se_ref, znt_bf_ref, acc_ref):
    step = pl.program_id(0)

    @pl.when(step == 0)
    def _init():
        zp = zp_ref[...]
        nrm = jnp.sqrt(jnp.sum(zp * zp, axis=1, keepdims=True))
        znt = zp * (_LOG2E / TEMP) / jnp.maximum(nrm, 1e-12)
        znt_ref[...] = znt
        znt_bf_ref[...] = znt.astype(jnp.bfloat16)
        acc_ref[...] = jnp.zeros_like(acc_ref)

    znt_bf = znt_bf_ref[...]
    offs = []
    off = 0
    while off < _TN:
        offs.append((off, min(256, _TN - off)))
        off += 256

    def _sim(o, w):
        return lax.dot_general(znt_bf, mem_ref[o:o + w, :].astype(jnp.bfloat16),
                               (((1,), (1,)), ((), ())),
                               preferred_element_type=jnp.float32)

    def _consume(s, w):
        e = jnp.exp2(s)
        a = e[:, 0:128]
        if w > 128:
            b = e[:, 128:w]
            if w < 256:
                b = jnp.pad(b, ((0, 0), (0, 256 - w)))
            a = a + b
        return a

    t = jnp.zeros((BATCH, 128), dtype=jnp.float32)
    prev = None
    for (o, w) in offs:
        s = _sim(o, w)
        if prev is not None:
            t = t + _consume(*prev)
        prev = (s, w)
    t = t + _consume(*prev)
    acc_ref[...] += t

    @pl.when(step == _NT - 1)
    def _fin():
        se_ref[...] = jnp.sum(acc_ref[...], axis=1, keepdims=True)


_stream = pl.pallas_call(
    _stream_body,
    grid=(_NT,),
    in_specs=[
        pl.BlockSpec((BATCH, NPC_DIM), lambda i: (0, 0)),
        pl.BlockSpec((_TN, NPC_DIM), lambda i: (i, 0)),
    ],
    out_specs=[
        pl.BlockSpec((BATCH, NPC_DIM), lambda i: (0, 0)),
        pl.BlockSpec((BATCH, 1), lambda i: (0, 0)),
    ],
    out_shape=[
        jax.ShapeDtypeStruct((BATCH, NPC_DIM), jnp.float32),
        jax.ShapeDtypeStruct((BATCH, 1), jnp.float32),
    ],
    scratch_shapes=[pltpu.VMEM((BATCH, NPC_DIM), jnp.bfloat16),
                    pltpu.VMEM((BATCH, 128), jnp.float32)],
    compiler_params=pltpu.CompilerParams(
        dimension_semantics=("arbitrary",),
        vmem_limit_bytes=112 * 1024 * 1024,
    ),
)


def _loss_body(znt_ref, se_ref, flg_ref, rs_ref, rl_ref, rn_ref,
               inst_ref, anch_ref):
    znt = znt_ref[...]
    inv = 1.0 / se_ref[...]
    p_self = jnp.exp2(jnp.sum(znt * rs_ref[...], axis=1, keepdims=True)) * inv
    p_ln = jnp.exp2(jnp.sum(znt * rl_ref[...], axis=1, keepdims=True)) * inv
    p_nb = jnp.exp2(jnp.sum(znt * rn_ref[...], axis=1, keepdims=True)) * inv
    flg = flg_ref[...]
    inst_terms = jnp.log(p_self + p_ln + CONST)
    anch_terms = jnp.log(p_self + p_nb + p_ln + CONST)
    scale = -2.0 / BATCH
    inst = jnp.sum(jnp.where(flg < 0, inst_terms, 0.0)) * scale
    anch = jnp.sum(jnp.where(flg >= 0, anch_terms, 0.0)) * scale
    inst_ref[...] = inst[None, None]
    anch_ref[...] = anch[None, None]


_loss = pl.pallas_call(
    _loss_body,
    out_shape=[
        jax.ShapeDtypeStruct((1, 1), jnp.float32),
        jax.ShapeDtypeStruct((1, 1), jnp.float32),
    ],
)


def kernel(zp, index, local_neighbor_indices, memory, flag, neighbors):
    flags, rows_self, rows_ln, rows_nb = _get_sc_gather()(
        index, local_neighbor_indices, memory, flag, neighbors)
    znt, se = _stream(zp, memory)
    inst, anch = _loss(znt, se, flags.reshape(BATCH, 1),
                       rows_self, rows_ln, rows_nb)
    return (inst[0, 0], anch[0, 0])


# speedup vs baseline: 1.0218x; 1.0218x over previous
import functools

import jax
import jax.numpy as jnp
from jax import lax
from jax.experimental import pallas as pl
from jax.experimental.pallas import tpu as pltpu
from jax.experimental.pallas import tpu_sc as plsc

N_SAMPLES = 100000
NPC_DIM = 128
BATCH = 4096
TEMP = 0.07
CONST = 1e-12

_NC = 2
_NS = 16
_NW = _NC * _NS
_BPW = BATCH // _NW

_TN = 10000
_NT = N_SAMPLES // _TN


@functools.cache
def _get_sc_gather():
    mesh = plsc.VectorSubcoreMesh(core_axis_name="c", subcore_axis_name="s")

    @functools.partial(
        pl.kernel,
        mesh=mesh,
        out_type=[
            jax.ShapeDtypeStruct((BATCH,), jnp.int32),
            jax.ShapeDtypeStruct((BATCH, NPC_DIM), jnp.float32),
            jax.ShapeDtypeStruct((BATCH, NPC_DIM), jnp.float32),
            jax.ShapeDtypeStruct((BATCH, NPC_DIM), jnp.float32),
        ],
        scratch_types=[
            pltpu.VMEM((_BPW,), jnp.int32),
            pltpu.VMEM((_BPW,), jnp.int32),
            pltpu.VMEM((_BPW,), jnp.int32),
            pltpu.VMEM((_BPW,), jnp.int32),
            pltpu.VMEM((_BPW, NPC_DIM), jnp.float32),
            pltpu.VMEM((_BPW, NPC_DIM), jnp.float32),
            pltpu.VMEM((_BPW, NPC_DIM), jnp.float32),
            pltpu.SemaphoreType.DMA,
            pltpu.SemaphoreType.DMA,
        ],
    )
    def _sc_gather(index_hbm, lni_hbm, memory_hbm, flag_hbm, neighbors_hbm,
                   flags_out, rows_self_out, rows_ln_out, rows_nb_out,
                   idx_v, lni_v, nbi_v, flg_v, rs_v, rl_v, rn_v, sem_a, sem_b):
        wid = lax.axis_index("s") * _NC + lax.axis_index("c")
        base = wid * _BPW
        pltpu.sync_copy(index_hbm.at[pl.ds(base, _BPW)], idx_v)
        pltpu.sync_copy(lni_hbm.at[pl.ds(base, _BPW)], lni_v)
        c_nb = pltpu.async_copy(neighbors_hbm.at[idx_v], nbi_v, sem_a)
        c_fl = pltpu.async_copy(flag_hbm.at[idx_v], flg_v, sem_b)
        c_rs = pltpu.async_copy(memory_hbm.at[idx_v], rs_v, sem_b)
        c_rl = pltpu.async_copy(memory_hbm.at[lni_v], rl_v, sem_b)
        c_nb.wait()
        c_rn = pltpu.async_copy(memory_hbm.at[nbi_v], rn_v, sem_b)
        c_fl.wait()
        c_rs.wait()
        c_rl.wait()
        c_rn.wait()
        pltpu.sync_copy(flg_v, flags_out.at[pl.ds(base, _BPW)])
        pltpu.sync_copy(rs_v, rows_self_out.at[pl.ds(base, _BPW)])
        pltpu.sync_copy(rl_v, rows_ln_out.at[pl.ds(base, _BPW)])
        pltpu.sync_copy(rn_v, rows_nb_out.at[pl.ds(base, _BPW)])

    return _sc_gather


_LOG2E = 1.4426950408889634


def _stream_body(zp_ref, mem_ref, znt_ref, se_ref, znt_bf_ref, acc_ref):
    step = pl.program_id(0)

    @pl.when(step == 0)
    def _init():
        zp = zp_ref[...]
        nrm = jnp.sqrt(jnp.sum(zp * zp, axis=1, keepdims=True))
        znt = zp * (_LOG2E / TEMP) / jnp.maximum(nrm, 1e-12)
        znt_ref[...] = znt
        znt_bf_ref[...] = znt.astype(jnp.bfloat16)
        acc_ref[...] = jnp.zeros_like(acc_ref)

    znt_bf = znt_bf_ref[...]
    offs = []
    off = 0
    while off < _TN:
        offs.append((off, min(256, _TN - off)))
        off += 256

    def _sim(o, w):
        return lax.dot_general(znt_bf, mem_ref[o:o + w, :].astype(jnp.bfloat16),
                               (((1,), (1,)), ((), ())),
                               preferred_element_type=jnp.float32)

    def _consume(s, w):
        e = jnp.exp2(s)
        a = e[:, 0:min(128, w)]
        if w < 128:
            a = jnp.pad(a, ((0, 0), (0, 128 - w)))
        elif w > 128:
            b = e[:, 128:w]
            if w < 256:
                b = jnp.pad(b, ((0, 0), (0, 256 - w)))
            a = a + b
        return a

    t = jnp.zeros((BATCH, 128), dtype=jnp.float32)
    prev = None
    for (o, w) in offs:
        s = _sim(o, w)
        if prev is not None:
            t = t + _consume(*prev)
        prev = (s, w)
    t = t + _consume(*prev)
    acc_ref[...] += t

    @pl.when(step == _NT - 1)
    def _fin():
        se_ref[...] = jnp.sum(acc_ref[...], axis=1, keepdims=True)


_stream = pl.pallas_call(
    _stream_body,
    grid=(_NT,),
    in_specs=[
        pl.BlockSpec((BATCH, NPC_DIM), lambda i: (0, 0)),
        pl.BlockSpec((_TN, NPC_DIM), lambda i: (i, 0)),
    ],
    out_specs=[
        pl.BlockSpec((BATCH, NPC_DIM), lambda i: (0, 0)),
        pl.BlockSpec((BATCH, 1), lambda i: (0, 0)),
    ],
    out_shape=[
        jax.ShapeDtypeStruct((BATCH, NPC_DIM), jnp.float32),
        jax.ShapeDtypeStruct((BATCH, 1), jnp.float32),
    ],
    scratch_shapes=[pltpu.VMEM((BATCH, NPC_DIM), jnp.bfloat16),
                    pltpu.VMEM((BATCH, 128), jnp.float32)],
    compiler_params=pltpu.CompilerParams(
        dimension_semantics=("arbitrary",),
        vmem_limit_bytes=112 * 1024 * 1024,
    ),
)


def _loss_body(znt_ref, se_ref, flg_ref, rs_ref, rl_ref, rn_ref,
               inst_ref, anch_ref):
    znt = znt_ref[...]
    inv = 1.0 / se_ref[...]
    p_self = jnp.exp2(jnp.sum(znt * rs_ref[...], axis=1, keepdims=True)) * inv
    p_ln = jnp.exp2(jnp.sum(znt * rl_ref[...], axis=1, keepdims=True)) * inv
    p_nb = jnp.exp2(jnp.sum(znt * rn_ref[...], axis=1, keepdims=True)) * inv
    flg = flg_ref[...]
    inst_terms = jnp.log(p_self + p_ln + CONST)
    anch_terms = jnp.log(p_self + p_nb + p_ln + CONST)
    scale = -2.0 / BATCH
    inst = jnp.sum(jnp.where(flg < 0, inst_terms, 0.0)) * scale
    anch = jnp.sum(jnp.where(flg >= 0, anch_terms, 0.0)) * scale
    inst_ref[...] = inst[None, None]
    anch_ref[...] = anch[None, None]


_loss = pl.pallas_call(
    _loss_body,
    out_shape=[
        jax.ShapeDtypeStruct((1, 1), jnp.float32),
        jax.ShapeDtypeStruct((1, 1), jnp.float32),
    ],
)


def kernel(zp, index, local_neighbor_indices, memory, flag, neighbors):
    flags, rows_self, rows_ln, rows_nb = _get_sc_gather()(
        index, local_neighbor_indices, memory, flag, neighbors)
    znt, se = _stream(zp, memory)
    inst, anch = _loss(znt, se, flags.reshape(BATCH, 1),
                       rows_self, rows_ln, rows_nb)
    return (inst[0, 0], anch[0, 0])
